# baseline (speedup 1.0000x reference)
import functools

import jax
import jax.numpy as jnp
from jax import lax
from jax.experimental import pallas as pl
from jax.experimental.pallas import tpu as pltpu
from jax.experimental.pallas import tpu_sc as plsc

VOCAB = 1000000
D = 32
B = 16384

G = 8
NC = 2
NS = 16
NW = NC * NS
BPW = B // NW
L = 16

_mesh = plsc.VectorSubcoreMesh(core_axis_name="c", subcore_axis_name="s")


@functools.partial(
    pl.kernel,
    mesh=_mesh,
    out_type=jax.ShapeDtypeStruct((D, B), jnp.float32),
    compiler_params=pltpu.CompilerParams(needs_layout_passes=False),
    scratch_types=[
        pltpu.VMEM((BPW,), jnp.int32),
        pltpu.VMEM((BPW // G, G, D), jnp.float32),
        pltpu.VMEM((D, 128), jnp.float32),
        pltpu.SemaphoreType.DMA,
    ],
)
def _sc_gather(idx_hbm, tbl_hbm, outT_hbm, idx_v, rows_v, slab_v, sem):
    wid = lax.axis_index("s") * NC + lax.axis_index("c")
    base = wid * BPW
    pltpu.sync_copy(idx_hbm.at[pl.ds(base, BPW)], idx_v)

    for b0 in range(0, BPW, L):
        v = idx_v[pl.ds(b0, L)]
        for l in range(L):
            idx = v[l]
            b = b0 + l
            pltpu.async_copy(
                tbl_hbm.at[idx >> 3, idx & 7], rows_v.at[b // G, b % G], sem
            )
    pltpu.make_async_copy(
        tbl_hbm.at[pl.ds(0, BPW // G)], rows_v, sem
    ).wait()
    lane = lax.iota(jnp.int32, L)
    for k in range(BPW // 128):
        for r in range(D):
            rv = jnp.full((L,), r, jnp.int32)
            for i in range(128 // L):
                b0 = k * 128 + i * L
                bvec = lane + b0
                slab_v[r, pl.ds(i * L, L)] = plsc.load_gather(
                    rows_v, [bvec >> 3, bvec & (G - 1), rv]
                )
        pltpu.sync_copy(
            slab_v, outT_hbm.at[:, pl.ds(base + k * 128, 128)]
        )


def kernel(down_ID, table):
    idx = down_ID.astype(jnp.int32)
    tbl = table.reshape(VOCAB // G, G, D)
    return _sc_gather(idx, tbl).T


# speedup vs baseline: 1.0447x; 1.0447x over previous
import functools

import jax
import jax.numpy as jnp
from jax import lax
from jax.experimental import pallas as pl
from jax.experimental.pallas import tpu as pltpu
from jax.experimental.pallas import tpu_sc as plsc

VOCAB = 1000000
D = 32
B = 16384

G = 8
NC = 2
NS = 16
NW = NC * NS
BPW = B // NW

_mesh = plsc.VectorSubcoreMesh(core_axis_name="c", subcore_axis_name="s")


@functools.partial(
    pl.kernel,
    mesh=_mesh,
    out_type=jax.ShapeDtypeStruct((B, D), jnp.float32),
    compiler_params=pltpu.CompilerParams(needs_layout_passes=False),
    scratch_types=[
        pltpu.VMEM((BPW,), jnp.int32),
        pltpu.VMEM((BPW, D), jnp.float32),
        pltpu.SemaphoreType.DMA,
    ],
)
def _sc_gather(idx_hbm, tbl_hbm, out_hbm, idx_v, rows_v, sem):
    wid = lax.axis_index("s") * NC + lax.axis_index("c")
    base = wid * BPW
    pltpu.sync_copy(idx_hbm.at[pl.ds(base, BPW)], idx_v)

    for b0 in range(0, BPW, 16):
        v = idx_v[pl.ds(b0, 16)]
        for l in range(16):
            idx = v[l]
            pltpu.async_copy(
                tbl_hbm.at[idx >> 3, idx & 7], rows_v.at[b0 + l], sem
            )
    pltpu.make_async_copy(
        out_hbm.at[pl.ds(base, BPW)], rows_v, sem
    ).wait()
    pltpu.sync_copy(rows_v, out_hbm.at[pl.ds(base, BPW)])


def kernel(down_ID, table):
    idx = down_ID.astype(jnp.int32)
    tbl = table.reshape(VOCAB // G, G, D)
    return _sc_gather(idx, tbl)
